# Initial kernel scaffold; baseline (speedup 1.0000x reference)
import jax
import jax.numpy as jnp
from jax.experimental import pallas as pl


def kernel(xyz1, xyz2, points1, points2, W_fuse, b_fuse, g_fuse, be_fuse, W1, b1, g1, be1, W2, b2, g2, be2):
    raise NotImplementedError("write your pallas kernel here")


# speedup vs baseline: 14.3510x; 14.3510x over previous
import functools

import jax
import jax.numpy as jnp
from jax.experimental import pallas as pl


def _dot(a, b, dims):
    return jax.lax.dot_general(a, b, (dims, ((), ())),
                               preferred_element_type=jnp.float32)


def _topk_kernel(S, xyz1_ref, xyz2t_ref, idx_ref, w_ref):
    x = xyz1_ref[0]
    q = xyz2t_ref[0]
    nb = x.shape[0]
    cross = jax.lax.dot_general(x.astype(jnp.bfloat16), q.astype(jnp.bfloat16),
                                (((1,), (0,)), ((), ())),
                                preferred_element_type=jnp.float32)
    xs = (x[:, 0:1] * x[:, 0:1] + x[:, 1:2] * x[:, 1:2]) + x[:, 2:3] * x[:, 2:3]
    qs = (q[0:1, :] * q[0:1, :] + q[1:2, :] * q[1:2, :]) + q[2:3, :] * q[2:3, :]
    d = (-2.0 * cross + xs) + qs
    iota = jax.lax.broadcasted_iota(jnp.int32, (nb, S), 1)
    idxs, vals = [], []
    for k in range(3):
        mval = jnp.min(d, axis=1, keepdims=True)
        am = jnp.min(jnp.where(d == mval, iota, S), axis=1, keepdims=True)
        idxs.append(am)
        vals.append(mval)
        if k < 2:
            d = jnp.where(iota == am, jnp.float32(jnp.inf), d)
    d3 = jnp.concatenate(vals, axis=1)
    recip = 1.0 / (d3 + 1e-8)
    w = recip / jnp.sum(recip, axis=1, keepdims=True)
    idx_ref[...] = jnp.concatenate(idxs, axis=1)
    w_ref[...] = w


def _table_kernel(p2_ref, Wc2_ref, T_ref):
    T_ref[...] = _dot(p2_ref[0], Wc2_ref[...], (((0,), (1,))))


def _fuse_kernel(S, idx_ref, w_ref, p1_ref, T_ref, Wc1_ref, s1_ref, st_ref):
    b = pl.program_id(0)
    j = pl.program_id(1)
    idx = idx_ref[...]
    w = w_ref[...]
    nb = idx.shape[0]
    iota = jax.lax.broadcasted_iota(jnp.int32, (nb, S), 1)
    oh = jnp.zeros((nb, S), jnp.float32)
    for k in range(3):
        oh = oh + jnp.where(iota == idx[:, k:k + 1], w[:, k:k + 1], 0.0)
    s1 = _dot(oh, T_ref[...], (((1,), (0,))))
    s1 = s1 + _dot(p1_ref[0].astype(jnp.bfloat16),
                   Wc1_ref[...].astype(jnp.bfloat16), (((0,), (1,))))
    s1_ref[...] = s1

    @pl.when((b == 0) & (j == 0))
    def _():
        st_ref[...] = jnp.zeros_like(st_ref)
    st_ref[0:1, :] += jnp.sum(s1, axis=0, keepdims=True)
    st_ref[1:2, :] += jnp.sum(s1 * s1, axis=0, keepdims=True)


def _mid_kernel(store_x, s_ref, sc_ref, sh_ref, W_ref, *out_refs):
    if store_x:
        x_ref, s2_ref, st_ref = out_refs
    else:
        s2_ref, st_ref = out_refs
    x = jnp.maximum(s_ref[...] * sc_ref[...] + sh_ref[...], 0.0)
    s2 = _dot(x.astype(jnp.bfloat16), W_ref[...].astype(jnp.bfloat16),
              (((1,), (1,))))
    if store_x:
        x_ref[...] = x
    s2_ref[...] = s2

    @pl.when(pl.program_id(0) == 0)
    def _():
        st_ref[...] = jnp.zeros_like(st_ref)
    st_ref[0:1, :] += jnp.sum(s2, axis=0, keepdims=True)
    st_ref[1:2, :] += jnp.sum(s2 * s2, axis=0, keepdims=True)


def _final_kernel(s3_ref, x_ref, sc_ref, sh_ref, o_ref):
    y = s3_ref[...] * sc_ref[...] + sh_ref[...] + x_ref[...]
    o_ref[0] = jnp.maximum(y, 0.0).T


def _stats_to_scale_shift(st, nt, g, be, eps):
    mean = st[0] / nt
    var = st[1] / nt - mean * mean
    scale = g / jnp.sqrt(var + eps)
    shift = be - mean * scale
    return scale[None, :], shift[None, :]


def kernel(xyz1, xyz2, points1, points2, W_fuse, b_fuse, g_fuse, be_fuse,
           W1, b1, g1, be1, W2, b2, g2, be2):
    B, N, _ = xyz1.shape
    S = xyz2.shape[1]
    C1 = points1.shape[1]
    C2 = points2.shape[1]
    CO = W_fuse.shape[0]
    NT = B * N
    nb = 512
    NB = N // nb
    f32 = jnp.float32

    xyz2t = jnp.transpose(xyz2, (0, 2, 1))
    Wc1 = W_fuse[:, :C1]
    Wc2 = W_fuse[:, C1:]

    idx, w = pl.pallas_call(
        functools.partial(_topk_kernel, S),
        grid=(B, NB),
        in_specs=[
            pl.BlockSpec((1, nb, 3), lambda b, j: (b, j, 0)),
            pl.BlockSpec((1, 3, S), lambda b, j: (b, 0, 0)),
        ],
        out_specs=[
            pl.BlockSpec((nb, 3), lambda b, j: (b * NB + j, 0)),
            pl.BlockSpec((nb, 3), lambda b, j: (b * NB + j, 0)),
        ],
        out_shape=[
            jax.ShapeDtypeStruct((NT, 3), jnp.int32),
            jax.ShapeDtypeStruct((NT, 3), f32),
        ],
    )(xyz1, xyz2t)

    T = pl.pallas_call(
        _table_kernel,
        grid=(B,),
        in_specs=[
            pl.BlockSpec((1, C2, S), lambda b: (b, 0, 0)),
            pl.BlockSpec((CO, C2), lambda b: (0, 0)),
        ],
        out_specs=pl.BlockSpec((S, CO), lambda b: (b, 0)),
        out_shape=jax.ShapeDtypeStruct((B * S, CO), f32),
    )(points2, Wc2)

    s1, st1 = pl.pallas_call(
        functools.partial(_fuse_kernel, S),
        grid=(B, NB),
        in_specs=[
            pl.BlockSpec((nb, 3), lambda b, j: (b * NB + j, 0)),
            pl.BlockSpec((nb, 3), lambda b, j: (b * NB + j, 0)),
            pl.BlockSpec((1, C1, nb), lambda b, j: (b, 0, j)),
            pl.BlockSpec((S, CO), lambda b, j: (b, 0)),
            pl.BlockSpec((CO, C1), lambda b, j: (0, 0)),
        ],
        out_specs=[
            pl.BlockSpec((nb, CO), lambda b, j: (b * NB + j, 0)),
            pl.BlockSpec((8, CO), lambda b, j: (0, 0)),
        ],
        out_shape=[
            jax.ShapeDtypeStruct((NT, CO), f32),
            jax.ShapeDtypeStruct((8, CO), f32),
        ],
    )(idx, w, points1, T, Wc1)

    sc1, sh1 = _stats_to_scale_shift(st1, NT, g_fuse, be_fuse, 1e-5)

    NBT = NT // nb
    x, s2, st2 = pl.pallas_call(
        functools.partial(_mid_kernel, True),
        grid=(NBT,),
        in_specs=[
            pl.BlockSpec((nb, CO), lambda i: (i, 0)),
            pl.BlockSpec((1, CO), lambda i: (0, 0)),
            pl.BlockSpec((1, CO), lambda i: (0, 0)),
            pl.BlockSpec((CO, CO), lambda i: (0, 0)),
        ],
        out_specs=[
            pl.BlockSpec((nb, CO), lambda i: (i, 0)),
            pl.BlockSpec((nb, CO), lambda i: (i, 0)),
            pl.BlockSpec((8, CO), lambda i: (0, 0)),
        ],
        out_shape=[
            jax.ShapeDtypeStruct((NT, CO), f32),
            jax.ShapeDtypeStruct((NT, CO), f32),
            jax.ShapeDtypeStruct((8, CO), f32),
        ],
    )(s1, sc1, sh1, W1)

    sc2, sh2 = _stats_to_scale_shift(st2, NT, g1, be1, 1e-5)

    s3, st3 = pl.pallas_call(
        functools.partial(_mid_kernel, False),
        grid=(NBT,),
        in_specs=[
            pl.BlockSpec((nb, CO), lambda i: (i, 0)),
            pl.BlockSpec((1, CO), lambda i: (0, 0)),
            pl.BlockSpec((1, CO), lambda i: (0, 0)),
            pl.BlockSpec((CO, CO), lambda i: (0, 0)),
        ],
        out_specs=[
            pl.BlockSpec((nb, CO), lambda i: (i, 0)),
            pl.BlockSpec((8, CO), lambda i: (0, 0)),
        ],
        out_shape=[
            jax.ShapeDtypeStruct((NT, CO), f32),
            jax.ShapeDtypeStruct((8, CO), f32),
        ],
    )(s2, sc2, sh2, W2)

    sc3, sh3 = _stats_to_scale_shift(st3, NT, g2, be2, 1e-5)

    out = pl.pallas_call(
        _final_kernel,
        grid=(B, NB),
        in_specs=[
            pl.BlockSpec((nb, CO), lambda b, j: (b * NB + j, 0)),
            pl.BlockSpec((nb, CO), lambda b, j: (b * NB + j, 0)),
            pl.BlockSpec((1, CO), lambda b, j: (0, 0)),
            pl.BlockSpec((1, CO), lambda b, j: (0, 0)),
        ],
        out_specs=pl.BlockSpec((1, CO, nb), lambda b, j: (b, 0, j)),
        out_shape=jax.ShapeDtypeStruct((B, CO, N), f32),
    )(s3, x, sc3, sh3)

    return out
